# baseline (speedup 1.0000x reference)
import functools

import jax
import jax.numpy as jnp
from jax import lax
from jax.experimental import pallas as pl
from jax.experimental.pallas import tpu as pltpu
from jax.experimental.pallas import tpu_sc as plsc

_NE = 100000
_D = 64
_B = 16384
_L = 16
_NC = 2
_NS = 16
_NW = _NC * _NS
_BPW = _B // _NW
_C = 128
_NCHUNK = _BPW // _C
_G = _D // _L

_mesh = plsc.VectorSubcoreMesh(core_axis_name="c", subcore_axis_name="s")


@functools.partial(
    pl.kernel,
    mesh=_mesh,
    compiler_params=pltpu.CompilerParams(
        needs_layout_passes=False, use_tc_tiling_on_sc=True),
    out_type=[
        jax.ShapeDtypeStruct((_B * _L,), jnp.float32),
        jax.ShapeDtypeStruct((_NW, _L), jnp.float32),
    ],
    scratch_types=[
        pltpu.SMEM((3 * _BPW,), jnp.int32),
        pltpu.VMEM((_BPW,), jnp.int32),
        pltpu.VMEM((_BPW,), jnp.int32),
        pltpu.VMEM((_BPW,), jnp.int32),
        pltpu.VMEM_SHARED((_NS, 3 * _BPW), jnp.int32),
        pltpu.VMEM((_C * _D,), jnp.float32),
        pltpu.VMEM((_C * _D,), jnp.float32),
        pltpu.VMEM((_C * _D,), jnp.float32),
        pltpu.VMEM((_C * _D,), jnp.float32),
        pltpu.VMEM((_C * _D,), jnp.float32),
        pltpu.VMEM((_C * _D,), jnp.float32),
        pltpu.VMEM((_C * _L,), jnp.float32),
        pltpu.VMEM((_L,), jnp.float32),
        pltpu.SemaphoreType.DMA,
    ],
)
def _sc_score(heads, rels, tails, ent_re, ent_im, rel_re, rel_im,
              score_out, sq_out,
              idx_s, idx_vh, idx_vr, idx_vt, idx_sh,
              bhre, bhim, btre, btim, brre, brim,
              score_buf, sq_buf, sem):
    sid = lax.axis_index("s")
    wid = sid * _NC + lax.axis_index("c")
    base = wid * _BPW

    pltpu.sync_copy(heads.at[pl.ds(base, _BPW)], idx_vh)
    pltpu.sync_copy(rels.at[pl.ds(base, _BPW)], idx_vr)
    pltpu.sync_copy(tails.at[pl.ds(base, _BPW)], idx_vt)
    pltpu.sync_copy(idx_vh, idx_sh.at[sid, pl.ds(0, _BPW)])
    pltpu.sync_copy(idx_vr, idx_sh.at[sid, pl.ds(_BPW, _BPW)])
    pltpu.sync_copy(idx_vt, idx_sh.at[sid, pl.ds(2 * _BPW, _BPW)])
    pltpu.sync_copy(idx_sh.at[sid], idx_s)

    def chunk_body(ci, sq_carry):
        off = base + ci * _C
        loc = ci * _C

        def issue_body(i, _):
            h = idx_s[loc + i]
            r = idx_s[_BPW + loc + i]
            t = idx_s[2 * _BPW + loc + i]
            dst = pl.ds(i * _D, _D)
            pltpu.async_copy(ent_re.at[pl.ds(h * _D, _D)], bhre.at[dst], sem)
            pltpu.async_copy(ent_im.at[pl.ds(h * _D, _D)], bhim.at[dst], sem)
            pltpu.async_copy(ent_re.at[pl.ds(t * _D, _D)], btre.at[dst], sem)
            pltpu.async_copy(ent_im.at[pl.ds(t * _D, _D)], btim.at[dst], sem)
            pltpu.async_copy(rel_re.at[pl.ds(r * _D, _D)], brre.at[dst], sem)
            pltpu.async_copy(rel_im.at[pl.ds(r * _D, _D)], brim.at[dst], sem)
            return 0

        lax.fori_loop(0, _C, issue_body, 0)
        for buf in (bhre, bhim, btre, btim, brre, brim):
            pltpu.make_async_copy(
                ent_re.at[pl.ds(0, _C * _D)], buf, sem).wait()

        def tri_body(i, carry):
            sq1, sq2, sq3 = carry
            score16 = jnp.zeros((_L,), jnp.float32)
            for g in range(_G):
                sl = pl.ds(i * _D + g * _L, _L)
                hre = bhre[sl]
                him = bhim[sl]
                tre = btre[sl]
                tim = btim[sl]
                rre = brre[sl]
                rim = brim[sl]
                score16 = score16 + rre * (hre * tre + him * tim)
                score16 = score16 + rim * (hre * tim - him * tre)
                sq1 = sq1 + (hre * hre + him * him)
                sq2 = sq2 + (tre * tre + tim * tim)
                sq3 = sq3 + (rre * rre + rim * rim)
            score_buf[pl.ds(i * _L, _L)] = score16
            return (sq1, sq2, sq3)

        sq_carry = lax.fori_loop(0, _C, tri_body, sq_carry)
        pltpu.sync_copy(score_buf, score_out.at[pl.ds(off * _L, _C * _L)])
        return sq_carry

    zero = jnp.zeros((_L,), jnp.float32)
    sq1, sq2, sq3 = lax.fori_loop(0, _NCHUNK, chunk_body, (zero, zero, zero))
    sq_buf[...] = sq1 + sq2 + sq3
    pltpu.sync_copy(sq_buf, sq_out.at[wid])


def _tc_finish_body(part_ref, labels_ref, sq_ref, out_ref):
    part = part_ref[...]
    row = lax.broadcasted_iota(jnp.int32, (128, 8), 0)
    col = lax.broadcasted_iota(jnp.int32, (128, 8), 1)
    sel = (row // _L == col).astype(jnp.float32)
    score8 = -jax.lax.dot_general(
        part, sel, (((1,), (0,)), ((), ())),
        preferred_element_type=jnp.float32)
    x = score8 * labels_ref[...]
    sp = jnp.maximum(x, 0.0) + jnp.log(1.0 + jnp.exp(-jnp.abs(x)))
    regul = jnp.sum(sq_ref[...]) * (0.01 / (_B * _D))
    total = jnp.sum(sp) * (1.0 / _B) + regul
    out_ref[...] = jnp.broadcast_to(total, (1, 1))


def _tc_finish(part, labels8, sq):
    return pl.pallas_call(
        _tc_finish_body,
        out_shape=jax.ShapeDtypeStruct((1, 1), jnp.float32),
    )(part, labels8, sq)


def kernel(batch, labels, ent_re, ent_im, rel_re, rel_im):
    heads = batch[:, 0]
    rels = batch[:, 1]
    tails = batch[:, 2]
    part, sq = _sc_score(
        heads, rels, tails,
        ent_re.reshape(_NE * _D), ent_im.reshape(_NE * _D),
        rel_re.reshape(_NE * _D), rel_im.reshape(_NE * _D))
    loss = _tc_finish(
        part.reshape(_B // 8, 128), labels.reshape(_B // 8, 8), sq)
    return loss[0, 0]


# speedup vs baseline: 1.3557x; 1.3557x over previous
import functools

import jax
import jax.numpy as jnp
from jax import lax
from jax.experimental import pallas as pl
from jax.experimental.pallas import tpu as pltpu
from jax.experimental.pallas import tpu_sc as plsc

_D = 64
_B = 16384
_L = 16
_NC = 2
_NS = 16
_NW = _NC * _NS
_BPW = _B // _NW
_C = 64
_NCHUNK = _BPW // _C
_G = _D // _L

_mesh = plsc.VectorSubcoreMesh(core_axis_name="c", subcore_axis_name="s")

_ROWBUF = [pltpu.VMEM((_C, _D), jnp.float32)] * 12


@functools.partial(
    pl.kernel,
    mesh=_mesh,
    compiler_params=pltpu.CompilerParams(
        needs_layout_passes=False, use_tc_tiling_on_sc=True),
    out_type=[
        jax.ShapeDtypeStruct((_B * _L,), jnp.float32),
        jax.ShapeDtypeStruct((_NW, _L), jnp.float32),
    ],
    scratch_types=[
        pltpu.SMEM((3 * _BPW,), jnp.int32),
        pltpu.VMEM((_BPW,), jnp.int32),
        pltpu.VMEM((_BPW,), jnp.int32),
        pltpu.VMEM((_BPW,), jnp.int32),
        pltpu.VMEM_SHARED((_NS, 3 * _BPW), jnp.int32),
    ] + _ROWBUF + [
        pltpu.VMEM((_C * _L,), jnp.float32),
        pltpu.VMEM((_L,), jnp.float32),
        pltpu.SemaphoreType.DMA,
        pltpu.SemaphoreType.DMA,
    ],
)
def _sc_score(heads, rels, tails, ent_re, ent_im, rel_re, rel_im,
              score_out, sq_out,
              idx_s, idx_vh, idx_vr, idx_vt, idx_sh,
              ahre, ahim, atre, atim, arre, arim,
              bhre, bhim, btre, btim, brre, brim,
              score_buf, sq_buf, sem_a, sem_b):
    sid = lax.axis_index("s")
    wid = sid * _NC + lax.axis_index("c")
    base = wid * _BPW

    pltpu.sync_copy(heads.at[pl.ds(base, _BPW)], idx_vh)
    pltpu.sync_copy(rels.at[pl.ds(base, _BPW)], idx_vr)
    pltpu.sync_copy(tails.at[pl.ds(base, _BPW)], idx_vt)
    pltpu.sync_copy(idx_vh, idx_sh.at[sid, pl.ds(0, _BPW)])
    pltpu.sync_copy(idx_vr, idx_sh.at[sid, pl.ds(_BPW, _BPW)])
    pltpu.sync_copy(idx_vt, idx_sh.at[sid, pl.ds(2 * _BPW, _BPW)])
    pltpu.sync_copy(idx_sh.at[sid], idx_s)

    bufs_a = (ahre, ahim, atre, atim, arre, arim)
    bufs_b = (bhre, bhim, btre, btim, brre, brim)

    def issue(ci, bufs, sem):
        hre, him, tre, tim, rre, rim = bufs
        loc = ci * _C

        def issue_body(i, _):
            h = idx_s[loc + i]
            r = idx_s[_BPW + loc + i]
            t = idx_s[2 * _BPW + loc + i]
            pltpu.async_copy(ent_re.at[h], hre.at[i], sem)
            pltpu.async_copy(ent_im.at[h], him.at[i], sem)
            pltpu.async_copy(ent_re.at[t], tre.at[i], sem)
            pltpu.async_copy(ent_im.at[t], tim.at[i], sem)
            pltpu.async_copy(rel_re.at[r], rre.at[i], sem)
            pltpu.async_copy(rel_im.at[r], rim.at[i], sem)
            return 0

        lax.fori_loop(0, _C, issue_body, 0)

    def drain(bufs, sem):
        for buf in bufs:
            pltpu.make_async_copy(ent_re.at[pl.ds(0, _C)], buf, sem).wait()

    def compute(ci, bufs, sq_carry):
        hreb, himb, treb, timb, rreb, rimb = bufs
        off = base + ci * _C

        def tri_body(i, carry):
            sq1, sq2, sq3 = carry
            score16 = jnp.zeros((_L,), jnp.float32)
            for g in range(_G):
                sl = pl.ds(g * _L, _L)
                hre = hreb[i, sl]
                him = himb[i, sl]
                tre = treb[i, sl]
                tim = timb[i, sl]
                rre = rreb[i, sl]
                rim = rimb[i, sl]
                score16 = score16 + rre * (hre * tre + him * tim)
                score16 = score16 + rim * (hre * tim - him * tre)
                sq1 = sq1 + (hre * hre + him * him)
                sq2 = sq2 + (tre * tre + tim * tim)
                sq3 = sq3 + (rre * rre + rim * rim)
            score_buf[pl.ds(i * _L, _L)] = score16
            return (sq1, sq2, sq3)

        sq_carry = lax.fori_loop(0, _C, tri_body, sq_carry)
        pltpu.sync_copy(score_buf, score_out.at[pl.ds(off * _L, _C * _L)])
        return sq_carry

    zero = jnp.zeros((_L,), jnp.float32)
    sq = (zero, zero, zero)
    issue(0, bufs_a, sem_a)
    for ci in range(_NCHUNK):
        cur, csem = (bufs_a, sem_a) if ci % 2 == 0 else (bufs_b, sem_b)
        nxt, nsem = (bufs_b, sem_b) if ci % 2 == 0 else (bufs_a, sem_a)
        if ci + 1 < _NCHUNK:
            issue(ci + 1, nxt, nsem)
        drain(cur, csem)
        sq = compute(ci, cur, sq)
    sq_buf[...] = sq[0] + sq[1] + sq[2]
    pltpu.sync_copy(sq_buf, sq_out.at[wid])


def _tc_finish_body(part_ref, labels_ref, sq_ref, out_ref):
    part = part_ref[...]
    row = lax.broadcasted_iota(jnp.int32, (128, 8), 0)
    col = lax.broadcasted_iota(jnp.int32, (128, 8), 1)
    sel = (row // _L == col).astype(jnp.float32)
    score8 = -jax.lax.dot_general(
        part, sel, (((1,), (0,)), ((), ())),
        preferred_element_type=jnp.float32)
    x = score8 * labels_ref[...]
    sp = jnp.maximum(x, 0.0) + jnp.log(1.0 + jnp.exp(-jnp.abs(x)))
    regul = jnp.sum(sq_ref[...]) * (0.01 / (_B * _D))
    total = jnp.sum(sp) * (1.0 / _B) + regul
    out_ref[...] = jnp.broadcast_to(total, (1, 1))


def _tc_finish(part, labels8, sq):
    return pl.pallas_call(
        _tc_finish_body,
        out_shape=jax.ShapeDtypeStruct((1, 1), jnp.float32),
    )(part, labels8, sq)


def kernel(batch, labels, ent_re, ent_im, rel_re, rel_im):
    heads = batch[:, 0]
    rels = batch[:, 1]
    tails = batch[:, 2]
    part, sq = _sc_score(heads, rels, tails, ent_re, ent_im, rel_re, rel_im)
    loss = _tc_finish(
        part.reshape(_B // 8, 128), labels.reshape(_B // 8, 8), sq)
    return loss[0, 0]
